# Initial kernel scaffold; baseline (speedup 1.0000x reference)
import jax
import jax.numpy as jnp
from jax.experimental import pallas as pl


def kernel(x, edge_index, W_l, W_r, b):
    raise NotImplementedError("write your pallas kernel here")


# speedup vs baseline: 6.0088x; 6.0088x over previous
import functools

import jax
import jax.numpy as jnp
from jax import lax
from jax.experimental import pallas as pl
from jax.experimental.pallas import tpu as pltpu
from jax.experimental.pallas import tpu_sc as plsc

N = 10000
E = 320000
D = 128

NC = 2
NS = 16
NW = NC * NS
EW = E // NW
C = 80
NCHUNK = EW // C
RPS = 624
TAIL = N - RPS * NS


def _agg_body(x_hbm, src_hbm, dst_hbm, zf_hbm, zd_hbm, part_hbm, deg_hbm,
              src_v, dst_v, rows_v, ones_v, acc_sh, deg_sh, sem):
    cid = lax.axis_index("c")
    sid = lax.axis_index("s")
    wid = sid * NC + cid

    for i in range(C // 16):
        ones_v[pl.ds(i * 16, 16)] = jnp.full((16,), 1.0, dtype=jnp.float32)
    pltpu.sync_copy(zf_hbm.at[pl.ds(0, RPS)], acc_sh.at[pl.ds(sid * RPS, RPS)])

    @pl.when(sid == 0)
    def _():
        pltpu.sync_copy(zf_hbm.at[pl.ds(0, TAIL)],
                        acc_sh.at[pl.ds(RPS * NS, TAIL)])
        pltpu.sync_copy(zd_hbm, deg_sh)

    plsc.subcore_barrier()

    @pl.loop(0, NCHUNK)
    def _(t):
        base = wid * EW + t * C
        pltpu.sync_copy(src_hbm.at[pl.ds(base, C)], src_v)
        pltpu.sync_copy(dst_hbm.at[pl.ds(base, C)], dst_v)
        pltpu.async_copy(x_hbm.at[src_v], rows_v, sem).wait()
        pltpu.sync_copy(rows_v, acc_sh.at[dst_v], add=True)
        pltpu.sync_copy(ones_v, deg_sh.at[dst_v], add=True)

    plsc.subcore_barrier()

    pltpu.sync_copy(acc_sh.at[pl.ds(sid * RPS, RPS)],
                    part_hbm.at[cid, pl.ds(sid * RPS, RPS)])

    @pl.when(sid == 0)
    def _():
        pltpu.sync_copy(acc_sh.at[pl.ds(RPS * NS, TAIL)],
                        part_hbm.at[cid, pl.ds(RPS * NS, TAIL)])
        pltpu.sync_copy(deg_sh, deg_hbm.at[cid])


@functools.cache
def _make_aggregate():
    return pl.kernel(
        _agg_body,
        out_type=[
            jax.ShapeDtypeStruct((NC, N, D), jnp.float32),
            jax.ShapeDtypeStruct((NC, N), jnp.float32),
        ],
        mesh=plsc.VectorSubcoreMesh(core_axis_name="c", subcore_axis_name="s",
                                    num_cores=NC, num_subcores=NS),
        scratch_types=[
            pltpu.VMEM((C,), jnp.int32),
            pltpu.VMEM((C,), jnp.int32),
            pltpu.VMEM((C, D), jnp.float32),
            pltpu.VMEM((C,), jnp.float32),
            pltpu.VMEM_SHARED((N, D), jnp.float32),
            pltpu.VMEM_SHARED((N,), jnp.float32),
            pltpu.SemaphoreType.DMA,
        ],
    )


BR = 1000


def _tc_body(p0, p1, d0, d1, x, wl, wr, b, o):
    deg = jnp.maximum(d0[...] + d1[...], 1.0)
    aggr = (p0[...] + p1[...]) / deg
    o[...] = (jnp.dot(aggr, wl[...], preferred_element_type=jnp.float32)
              + jnp.dot(x[...], wr[...], preferred_element_type=jnp.float32)
              + b[...])


_combine = pl.pallas_call(
    _tc_body,
    grid=(N // BR,),
    in_specs=[
        pl.BlockSpec((BR, D), lambda i: (i, 0)),
        pl.BlockSpec((BR, D), lambda i: (i, 0)),
        pl.BlockSpec((BR, 1), lambda i: (i, 0)),
        pl.BlockSpec((BR, 1), lambda i: (i, 0)),
        pl.BlockSpec((BR, D), lambda i: (i, 0)),
        pl.BlockSpec((D, D), lambda i: (0, 0)),
        pl.BlockSpec((D, D), lambda i: (0, 0)),
        pl.BlockSpec((1, D), lambda i: (0, 0)),
    ],
    out_specs=pl.BlockSpec((BR, D), lambda i: (i, 0)),
    out_shape=jax.ShapeDtypeStruct((N, D), jnp.float32),
)


def kernel(x, edge_index, W_l, W_r, b):
    src = edge_index[0].astype(jnp.int32)
    dst = edge_index[1].astype(jnp.int32)
    zf = jnp.zeros((RPS, D), dtype=jnp.float32)
    zd = jnp.zeros((N,), dtype=jnp.float32)
    part, deg = _make_aggregate()(x, src, dst, zf, zd)
    return _combine(part[0], part[1],
                    deg[0].reshape(N, 1), deg[1].reshape(N, 1),
                    x, W_l.T, W_r.T, b.reshape(1, D))


# speedup vs baseline: 12.1242x; 2.0177x over previous
import functools

import jax
import jax.numpy as jnp
from jax import lax
from jax.experimental import pallas as pl
from jax.experimental.pallas import tpu as pltpu
from jax.experimental.pallas import tpu_sc as plsc

N = 10000
E = 320000
D = 128

NC = 2
NS = 16
NW = NC * NS
EW = E // NW
C = 80
NCHUNK = EW // C
RPS = 624
TAIL = N - RPS * NS


def _agg_body(x_hbm, src_hbm, dst_hbm, zf_hbm, zd_hbm, part_hbm, deg_hbm,
              src_all, dst_all, rows0_v, rows1_v, ones_v, acc_sh, deg_sh,
              g0, g1):
    cid = lax.axis_index("c")
    sid = lax.axis_index("s")
    wid = sid * NC + cid

    for i in range(C // 16):
        ones_v[pl.ds(i * 16, 16)] = jnp.full((16,), 1.0, dtype=jnp.float32)
    pltpu.sync_copy(zf_hbm.at[pl.ds(0, RPS)], acc_sh.at[pl.ds(sid * RPS, RPS)])

    @pl.when(sid == 0)
    def _():
        pltpu.sync_copy(zf_hbm.at[pl.ds(0, TAIL)],
                        acc_sh.at[pl.ds(RPS * NS, TAIL)])
        pltpu.sync_copy(zd_hbm, deg_sh)

    pltpu.sync_copy(src_hbm.at[pl.ds(wid * EW, EW)], src_all)
    pltpu.sync_copy(dst_hbm.at[wid], dst_all)
    plsc.subcore_barrier()

    pltpu.async_copy(x_hbm.at[src_all.at[pl.ds(0, C)]], rows0_v, g0)
    pltpu.async_copy(x_hbm.at[src_all.at[pl.ds(C, C)]], rows1_v, g1)

    def chunk_body(t, rows_v, sem):
        pltpu.make_async_copy(x_hbm.at[pl.ds(0, C)], rows_v, sem).wait()
        pltpu.sync_copy(rows_v, acc_sh.at[dst_all.at[t]], add=True)
        pltpu.sync_copy(ones_v, deg_sh.at[dst_all.at[t]], add=True)

        @pl.when(t + 2 < NCHUNK)
        def _():
            pltpu.async_copy(x_hbm.at[src_all.at[pl.ds((t + 2) * C, C)]],
                             rows_v, sem)

    @pl.loop(0, NCHUNK)
    def _(t):
        even = lax.rem(t, 2) == 0

        @pl.when(even)
        def _():
            chunk_body(t, rows0_v, g0)

        @pl.when(jnp.logical_not(even))
        def _():
            chunk_body(t, rows1_v, g1)

    plsc.subcore_barrier()

    pltpu.sync_copy(acc_sh.at[pl.ds(sid * RPS, RPS)],
                    part_hbm.at[cid, pl.ds(sid * RPS, RPS)])

    @pl.when(sid == 0)
    def _():
        pltpu.sync_copy(acc_sh.at[pl.ds(RPS * NS, TAIL)],
                        part_hbm.at[cid, pl.ds(RPS * NS, TAIL)])
        pltpu.sync_copy(deg_sh, deg_hbm.at[cid])


@functools.cache
def _make_aggregate():
    return pl.kernel(
        _agg_body,
        out_type=[
            jax.ShapeDtypeStruct((NC, N, D), jnp.float32),
            jax.ShapeDtypeStruct((NC, N), jnp.float32),
        ],
        mesh=plsc.VectorSubcoreMesh(core_axis_name="c", subcore_axis_name="s",
                                    num_cores=NC, num_subcores=NS),
        scratch_types=[
            pltpu.VMEM((EW,), jnp.int32),
            pltpu.VMEM((NCHUNK, C), jnp.int32),
            pltpu.VMEM((C, D), jnp.float32),
            pltpu.VMEM((C, D), jnp.float32),
            pltpu.VMEM((C,), jnp.float32),
            pltpu.VMEM_SHARED((N, D), jnp.float32),
            pltpu.VMEM_SHARED((N,), jnp.float32),
            pltpu.SemaphoreType.DMA,
            pltpu.SemaphoreType.DMA,
        ],
    )


BR = 1000


def _tc_body(p0, p1, d0, d1, x, wl, wr, b, o):
    deg = jnp.maximum(d0[...] + d1[...], 1.0)
    aggr = (p0[...] + p1[...]) / deg
    o[...] = (jnp.dot(aggr, wl[...], preferred_element_type=jnp.float32)
              + jnp.dot(x[...], wr[...], preferred_element_type=jnp.float32)
              + b[...])


_combine = pl.pallas_call(
    _tc_body,
    grid=(N // BR,),
    in_specs=[
        pl.BlockSpec((BR, D), lambda i: (i, 0)),
        pl.BlockSpec((BR, D), lambda i: (i, 0)),
        pl.BlockSpec((BR, 1), lambda i: (i, 0)),
        pl.BlockSpec((BR, 1), lambda i: (i, 0)),
        pl.BlockSpec((BR, D), lambda i: (i, 0)),
        pl.BlockSpec((D, D), lambda i: (0, 0)),
        pl.BlockSpec((D, D), lambda i: (0, 0)),
        pl.BlockSpec((1, D), lambda i: (0, 0)),
    ],
    out_specs=pl.BlockSpec((BR, D), lambda i: (i, 0)),
    out_shape=jax.ShapeDtypeStruct((N, D), jnp.float32),
)


def kernel(x, edge_index, W_l, W_r, b):
    src = edge_index[0].astype(jnp.int32)
    dst = edge_index[1].astype(jnp.int32).reshape(NW, NCHUNK, C)
    zf = jnp.zeros((RPS, D), dtype=jnp.float32)
    zd = jnp.zeros((N,), dtype=jnp.float32)
    part, deg = _make_aggregate()(x, src, dst, zf, zd)
    return _combine(part[0], part[1],
                    deg[0].reshape(N, 1), deg[1].reshape(N, 1),
                    x, W_l.T, W_r.T, b.reshape(1, D))


# speedup vs baseline: 12.2966x; 1.0142x over previous
import functools

import jax
import jax.numpy as jnp
from jax import lax
from jax.experimental import pallas as pl
from jax.experimental.pallas import tpu as pltpu
from jax.experimental.pallas import tpu_sc as plsc

N = 10000
E = 320000
D = 128

NC = 2
NS = 16
NW = NC * NS
EW = E // NW
C = 80
NCHUNK = EW // C
RPS = 624
TAIL = N - RPS * NS
ZR = 208


def _agg_body(x_hbm, src_hbm, dst_hbm, zf_hbm, zd_hbm, part_hbm, deg_hbm,
              src_all, dst_all, rows0_v, rows1_v, ones_v, acc_sh, deg_sh,
              g0, g1):
    cid = lax.axis_index("c")
    sid = lax.axis_index("s")
    wid = sid * NC + cid

    for i in range(C // 16):
        ones_v[pl.ds(i * 16, 16)] = jnp.full((16,), 1.0, dtype=jnp.float32)
    for j in range(RPS // ZR):
        pltpu.sync_copy(zf_hbm,
                        acc_sh.at[pl.ds(sid * RPS + j * ZR, ZR)])

    @pl.when(sid == 0)
    def _():
        pltpu.sync_copy(zf_hbm.at[pl.ds(0, TAIL)],
                        acc_sh.at[pl.ds(RPS * NS, TAIL)])
        pltpu.sync_copy(zd_hbm, deg_sh)

    pltpu.sync_copy(src_hbm.at[pl.ds(wid * EW, EW)], src_all)
    pltpu.sync_copy(dst_hbm.at[wid], dst_all)
    plsc.subcore_barrier()

    pltpu.async_copy(x_hbm.at[src_all.at[pl.ds(0, C)]], rows0_v, g0)
    pltpu.async_copy(x_hbm.at[src_all.at[pl.ds(C, C)]], rows1_v, g1)

    def chunk_body(t, rows_v, sem):
        pltpu.make_async_copy(x_hbm.at[pl.ds(0, C)], rows_v, sem).wait()
        pltpu.sync_copy(rows_v, acc_sh.at[dst_all.at[t]], add=True)
        pltpu.sync_copy(ones_v, deg_sh.at[dst_all.at[t]], add=True)

        @pl.when(t + 2 < NCHUNK)
        def _():
            pltpu.async_copy(x_hbm.at[src_all.at[pl.ds((t + 2) * C, C)]],
                             rows_v, sem)

    @pl.loop(0, NCHUNK)
    def _(t):
        even = lax.rem(t, 2) == 0

        @pl.when(even)
        def _():
            chunk_body(t, rows0_v, g0)

        @pl.when(jnp.logical_not(even))
        def _():
            chunk_body(t, rows1_v, g1)

    plsc.subcore_barrier()

    pltpu.sync_copy(acc_sh.at[pl.ds(sid * RPS, RPS)],
                    part_hbm.at[cid, pl.ds(sid * RPS, RPS)])

    @pl.when(sid == 0)
    def _():
        pltpu.sync_copy(acc_sh.at[pl.ds(RPS * NS, TAIL)],
                        part_hbm.at[cid, pl.ds(RPS * NS, TAIL)])
        pltpu.sync_copy(deg_sh, deg_hbm.at[cid])


@functools.cache
def _make_aggregate():
    return pl.kernel(
        _agg_body,
        out_type=[
            jax.ShapeDtypeStruct((NC, N, D), jnp.float32),
            jax.ShapeDtypeStruct((NC, N), jnp.float32),
        ],
        mesh=plsc.VectorSubcoreMesh(core_axis_name="c", subcore_axis_name="s",
                                    num_cores=NC, num_subcores=NS),
        scratch_types=[
            pltpu.VMEM((EW,), jnp.int32),
            pltpu.VMEM((NCHUNK, C), jnp.int32),
            pltpu.VMEM((C, D), jnp.float32),
            pltpu.VMEM((C, D), jnp.float32),
            pltpu.VMEM((C,), jnp.float32),
            pltpu.VMEM_SHARED((N, D), jnp.float32),
            pltpu.VMEM_SHARED((N,), jnp.float32),
            pltpu.SemaphoreType.DMA,
            pltpu.SemaphoreType.DMA,
        ],
    )


BR = 1000


def _tc_body(p0, p1, d0, d1, x, wl, wr, b, o):
    deg = jnp.maximum(d0[...] + d1[...], 1.0)
    aggr = (p0[...].reshape(BR, D) + p1[...].reshape(BR, D)) / deg
    dims = (((1,), (1,)), ((), ()))
    o[...] = (lax.dot_general(aggr, wl[...], dims,
                              preferred_element_type=jnp.float32)
              + lax.dot_general(x[...], wr[...], dims,
                                preferred_element_type=jnp.float32)
              + b[...])


_combine = pl.pallas_call(
    _tc_body,
    grid=(N // BR,),
    in_specs=[
        pl.BlockSpec((1, BR, D), lambda i: (0, i, 0)),
        pl.BlockSpec((1, BR, D), lambda i: (1, i, 0)),
        pl.BlockSpec((BR, 1), lambda i: (i, 0)),
        pl.BlockSpec((BR, 1), lambda i: (i, 0)),
        pl.BlockSpec((BR, D), lambda i: (i, 0)),
        pl.BlockSpec((D, D), lambda i: (0, 0)),
        pl.BlockSpec((D, D), lambda i: (0, 0)),
        pl.BlockSpec((1, D), lambda i: (0, 0)),
    ],
    out_specs=pl.BlockSpec((BR, D), lambda i: (i, 0)),
    out_shape=jax.ShapeDtypeStruct((N, D), jnp.float32),
)


def kernel(x, edge_index, W_l, W_r, b):
    src = edge_index[0].astype(jnp.int32)
    dst = edge_index[1].astype(jnp.int32).reshape(NW, NCHUNK, C)
    zf = jnp.zeros((ZR, D), dtype=jnp.float32)
    zd = jnp.zeros((N,), dtype=jnp.float32)
    part, deg = _make_aggregate()(x, src, dst, zf, zd)
    return _combine(part, part,
                    deg[0].reshape(N, 1), deg[1].reshape(N, 1),
                    x, W_l, W_r, b.reshape(1, D))


# speedup vs baseline: 14.2381x; 1.1579x over previous
import functools

import jax
import jax.numpy as jnp
from jax import lax
from jax.experimental import pallas as pl
from jax.experimental.pallas import tpu as pltpu
from jax.experimental.pallas import tpu_sc as plsc

N = 10000
E = 320000
D = 128

NC = 2
NS = 16
NW = NC * NS
EW = E // NW
C = 80
NCHUNK = EW // C
RPS = 624
TAIL = N - RPS * NS
ZR = 208


NBUF = 3
NIDX = 2 * NBUF


def _agg_body(x_hbm, src_hbm, dst_hbm, zf_hbm, zd_hbm, part_hbm, deg_hbm,
              dst_all, rows_bufs, idx_bufs, ones_v, acc_sh, deg_sh,
              gsems, isems):
    cid = lax.axis_index("c")
    sid = lax.axis_index("s")
    wid = sid * NC + cid

    for i in range(C // 16):
        ones_v[pl.ds(i * 16, 16)] = jnp.full((16,), 1.0, dtype=jnp.float32)
    for j in range(RPS // ZR):
        pltpu.sync_copy(zf_hbm,
                        acc_sh.at[pl.ds(sid * RPS + j * ZR, ZR)])

    @pl.when(sid == 0)
    def _():
        pltpu.sync_copy(zf_hbm.at[pl.ds(0, TAIL)],
                        acc_sh.at[pl.ds(RPS * NS, TAIL)])
        pltpu.sync_copy(zd_hbm, deg_sh)

    pltpu.sync_copy(dst_hbm.at[wid], dst_all)
    plsc.subcore_barrier()

    def load_idx(u, s):
        pltpu.async_copy(src_hbm.at[pl.ds(wid * EW + u * C, C)],
                         idx_bufs[s], isems[s])

    def start_gather(u, s, b):
        pltpu.make_async_copy(src_hbm.at[pl.ds(0, C)],
                              idx_bufs[s], isems[s]).wait()
        pltpu.async_copy(x_hbm.at[idx_bufs[s]], rows_bufs[b], gsems[b])

    for s in range(NIDX):
        load_idx(s, s)
    for b in range(NBUF):
        start_gather(b, b, b)

    @pl.loop(0, NCHUNK)
    def _(t):
        r = lax.rem(t, NIDX)
        for s6 in range(NIDX):
            @pl.when(r == s6)
            def _(s6=s6):
                b = s6 % NBUF
                pltpu.make_async_copy(x_hbm.at[pl.ds(0, C)],
                                      rows_bufs[b], gsems[b]).wait()
                pltpu.sync_copy(rows_bufs[b], acc_sh.at[dst_all.at[t]],
                                add=True)
                pltpu.sync_copy(ones_v, deg_sh.at[dst_all.at[t]], add=True)

                @pl.when(t + NBUF < NCHUNK)
                def _():
                    start_gather(t + NBUF, (s6 + NBUF) % NIDX, b)

                @pl.when(t + NIDX < NCHUNK)
                def _():
                    load_idx(t + NIDX, s6)

    plsc.subcore_barrier()

    pltpu.sync_copy(acc_sh.at[pl.ds(sid * RPS, RPS)],
                    part_hbm.at[cid, pl.ds(sid * RPS, RPS)])

    @pl.when(sid == 0)
    def _():
        pltpu.sync_copy(acc_sh.at[pl.ds(RPS * NS, TAIL)],
                        part_hbm.at[cid, pl.ds(RPS * NS, TAIL)])
        pltpu.sync_copy(deg_sh, deg_hbm.at[cid])


@functools.cache
def _make_aggregate():
    return pl.kernel(
        _agg_body,
        out_type=[
            jax.ShapeDtypeStruct((NC, N, D), jnp.float32),
            jax.ShapeDtypeStruct((NC, N), jnp.float32),
        ],
        mesh=plsc.VectorSubcoreMesh(core_axis_name="c", subcore_axis_name="s",
                                    num_cores=NC, num_subcores=NS),
        scratch_types=[
            pltpu.VMEM((NCHUNK, C), jnp.int32),
            [pltpu.VMEM((C, D), jnp.float32)] * NBUF,
            [pltpu.VMEM((C,), jnp.int32)] * NIDX,
            pltpu.VMEM((C,), jnp.float32),
            pltpu.VMEM_SHARED((N, D), jnp.float32),
            pltpu.VMEM_SHARED((N,), jnp.float32),
            [pltpu.SemaphoreType.DMA] * NBUF,
            [pltpu.SemaphoreType.DMA] * NIDX,
        ],
    )


BR = 1000


def _tc_body(p0, p1, d0, d1, x, wl, wr, b, o):
    deg = jnp.maximum(d0[...] + d1[...], 1.0)
    aggr = (p0[...].reshape(BR, D) + p1[...].reshape(BR, D)) / deg
    dims = (((1,), (1,)), ((), ()))
    o[...] = (lax.dot_general(aggr, wl[...], dims,
                              preferred_element_type=jnp.float32)
              + lax.dot_general(x[...], wr[...], dims,
                                preferred_element_type=jnp.float32)
              + b[...])


_combine = pl.pallas_call(
    _tc_body,
    grid=(N // BR,),
    in_specs=[
        pl.BlockSpec((1, BR, D), lambda i: (0, i, 0)),
        pl.BlockSpec((1, BR, D), lambda i: (1, i, 0)),
        pl.BlockSpec((BR, 1), lambda i: (i, 0)),
        pl.BlockSpec((BR, 1), lambda i: (i, 0)),
        pl.BlockSpec((BR, D), lambda i: (i, 0)),
        pl.BlockSpec((D, D), lambda i: (0, 0)),
        pl.BlockSpec((D, D), lambda i: (0, 0)),
        pl.BlockSpec((1, D), lambda i: (0, 0)),
    ],
    out_specs=pl.BlockSpec((BR, D), lambda i: (i, 0)),
    out_shape=jax.ShapeDtypeStruct((N, D), jnp.float32),
)


def kernel(x, edge_index, W_l, W_r, b):
    src = edge_index[0].astype(jnp.int32)
    dst = edge_index[1].astype(jnp.int32).reshape(NW, NCHUNK, C)
    zf = jnp.zeros((ZR, D), dtype=jnp.float32)
    zd = jnp.zeros((N,), dtype=jnp.float32)
    part, deg = _make_aggregate()(x, src, dst, zf, zd)
    return _combine(part, part,
                    deg[0].reshape(N, 1), deg[1].reshape(N, 1),
                    x, W_l, W_r, b.reshape(1, D))


# speedup vs baseline: 15.3328x; 1.0769x over previous
import functools

import jax
import jax.numpy as jnp
from jax import lax
from jax.experimental import pallas as pl
from jax.experimental.pallas import tpu as pltpu
from jax.experimental.pallas import tpu_sc as plsc

N = 10000
E = 320000
D = 128

NC = 2
NS = 16
NW = NC * NS
EW = E // NW
C = 80
NCHUNK = EW // C
RPS = 624
TAIL = N - RPS * NS
ZR = 208


NBUF = 3
NIDX = 2 * NBUF


def _agg_body(x_hbm, ei_hbm, zf_hbm, zd_hbm, part_hbm, deg_hbm,
              dst_all, rows_bufs, idx_bufs, ones_v, acc_sh, deg_sh,
              gsems, isems):
    cid = lax.axis_index("c")
    sid = lax.axis_index("s")
    wid = sid * NC + cid

    for i in range(C // 16):
        ones_v[pl.ds(i * 16, 16)] = jnp.full((16,), 1.0, dtype=jnp.float32)
    for j in range(RPS // ZR):
        pltpu.sync_copy(zf_hbm,
                        acc_sh.at[pl.ds(sid * RPS + j * ZR, ZR)])

    @pl.when(sid == 0)
    def _():
        pltpu.sync_copy(zf_hbm.at[pl.ds(0, TAIL)],
                        acc_sh.at[pl.ds(RPS * NS, TAIL)])
        pltpu.sync_copy(zd_hbm, deg_sh)

    pltpu.sync_copy(ei_hbm.at[1, wid], dst_all)
    plsc.subcore_barrier()

    def load_idx(u, s):
        pltpu.async_copy(ei_hbm.at[0, wid, u], idx_bufs[s], isems[s])

    def start_gather(u, s, b):
        pltpu.make_async_copy(ei_hbm.at[0, 0, 0],
                              idx_bufs[s], isems[s]).wait()
        pltpu.async_copy(x_hbm.at[idx_bufs[s]], rows_bufs[b], gsems[b])

    for s in range(NIDX):
        load_idx(s, s)
    for b in range(NBUF):
        start_gather(b, b, b)

    @pl.loop(0, NCHUNK)
    def _(t):
        r = lax.rem(t, NIDX)
        for s6 in range(NIDX):
            @pl.when(r == s6)
            def _(s6=s6):
                b = s6 % NBUF
                pltpu.make_async_copy(x_hbm.at[pl.ds(0, C)],
                                      rows_bufs[b], gsems[b]).wait()
                pltpu.sync_copy(rows_bufs[b], acc_sh.at[dst_all.at[t]],
                                add=True)
                pltpu.sync_copy(ones_v, deg_sh.at[dst_all.at[t]], add=True)

                @pl.when(t + NBUF < NCHUNK)
                def _():
                    start_gather(t + NBUF, (s6 + NBUF) % NIDX, b)

                @pl.when(t + NIDX < NCHUNK)
                def _():
                    load_idx(t + NIDX, s6)

    plsc.subcore_barrier()

    pltpu.sync_copy(acc_sh.at[pl.ds(sid * RPS, RPS)],
                    part_hbm.at[cid, pl.ds(sid * RPS, RPS)])

    @pl.when(sid == 0)
    def _():
        pltpu.sync_copy(acc_sh.at[pl.ds(RPS * NS, TAIL)],
                        part_hbm.at[cid, pl.ds(RPS * NS, TAIL)])
        pltpu.sync_copy(deg_sh, deg_hbm.at[cid])


@functools.cache
def _make_aggregate():
    return pl.kernel(
        _agg_body,
        out_type=[
            jax.ShapeDtypeStruct((NC, N, D), jnp.float32),
            jax.ShapeDtypeStruct((NC, N), jnp.float32),
        ],
        mesh=plsc.VectorSubcoreMesh(core_axis_name="c", subcore_axis_name="s",
                                    num_cores=NC, num_subcores=NS),
        scratch_types=[
            pltpu.VMEM((NCHUNK, C), jnp.int32),
            [pltpu.VMEM((C, D), jnp.float32)] * NBUF,
            [pltpu.VMEM((C,), jnp.int32)] * NIDX,
            pltpu.VMEM((C,), jnp.float32),
            pltpu.VMEM_SHARED((N, D), jnp.float32),
            pltpu.VMEM_SHARED((N,), jnp.float32),
            [pltpu.SemaphoreType.DMA] * NBUF,
            [pltpu.SemaphoreType.DMA] * NIDX,
        ],
    )


BR = 1000


def _tc_body(p, d0, d1, x, wl, wr, b, o):
    deg = jnp.maximum(d0[...] + d1[...], 1.0)
    aggr = (p[0] + p[1]) / deg
    dims = (((1,), (1,)), ((), ()))
    o[...] = (lax.dot_general(aggr, wl[...], dims,
                              preferred_element_type=jnp.float32)
              + lax.dot_general(x[...], wr[...], dims,
                                preferred_element_type=jnp.float32)
              + b[...])


_combine = pl.pallas_call(
    _tc_body,
    grid=(N // BR,),
    in_specs=[
        pl.BlockSpec((2, BR, D), lambda i: (0, i, 0)),
        pl.BlockSpec((BR, 1), lambda i: (i, 0)),
        pl.BlockSpec((BR, 1), lambda i: (i, 0)),
        pl.BlockSpec((BR, D), lambda i: (i, 0)),
        pl.BlockSpec((D, D), lambda i: (0, 0)),
        pl.BlockSpec((D, D), lambda i: (0, 0)),
        pl.BlockSpec((1, D), lambda i: (0, 0)),
    ],
    out_specs=pl.BlockSpec((BR, D), lambda i: (i, 0)),
    out_shape=jax.ShapeDtypeStruct((N, D), jnp.float32),
)


def kernel(x, edge_index, W_l, W_r, b):
    ei = edge_index.astype(jnp.int32).reshape(2, NW, NCHUNK, C)
    zf = jnp.zeros((ZR, D), dtype=jnp.float32)
    zd = jnp.zeros((N,), dtype=jnp.float32)
    part, deg = _make_aggregate()(x, ei, zf, zd)
    return _combine(part, deg[0].reshape(N, 1), deg[1].reshape(N, 1),
                    x, W_l, W_r, b.reshape(1, D))


# speedup vs baseline: 15.3384x; 1.0004x over previous
import functools

import jax
import jax.numpy as jnp
from jax import lax
from jax.experimental import pallas as pl
from jax.experimental.pallas import tpu as pltpu
from jax.experimental.pallas import tpu_sc as plsc

N = 10000
E = 320000
D = 128

NC = 2
NS = 16
NW = NC * NS
EW = E // NW
C = 80
NCHUNK = EW // C
RPS = 624
TAIL = N - RPS * NS
ZR = 208


NBUF = 4
NIDX = 2 * NBUF


def _agg_body(x_hbm, ei_hbm, zf_hbm, zd_hbm, part_hbm, deg_hbm,
              rows_bufs, src_bufs, dst_bufs, ones_v, acc_sh, deg_sh,
              gsems, isems, dsems):
    cid = lax.axis_index("c")
    sid = lax.axis_index("s")
    wid = sid * NC + cid

    for i in range(C // 16):
        ones_v[pl.ds(i * 16, 16)] = jnp.full((16,), 1.0, dtype=jnp.float32)
    for j in range(RPS // ZR):
        pltpu.sync_copy(zf_hbm,
                        acc_sh.at[pl.ds(sid * RPS + j * ZR, ZR)])

    @pl.when(sid == 0)
    def _():
        pltpu.sync_copy(zf_hbm.at[pl.ds(0, TAIL)],
                        acc_sh.at[pl.ds(RPS * NS, TAIL)])
        pltpu.sync_copy(zd_hbm, deg_sh)

    plsc.subcore_barrier()

    def load_idx(u, s):
        pltpu.async_copy(ei_hbm.at[0, wid, u], src_bufs[s], isems[s])
        pltpu.async_copy(ei_hbm.at[1, wid, u], dst_bufs[s], dsems[s])

    def start_gather(u, s, b):
        pltpu.make_async_copy(ei_hbm.at[0, 0, 0],
                              src_bufs[s], isems[s]).wait()
        pltpu.async_copy(x_hbm.at[src_bufs[s]], rows_bufs[b], gsems[b])

    for s in range(NIDX):
        load_idx(s, s)
    for b in range(NBUF):
        start_gather(b, b, b)

    @pl.loop(0, NCHUNK)
    def _(t):
        r = lax.rem(t, NIDX)
        for s8 in range(NIDX):
            @pl.when(r == s8)
            def _(s8=s8):
                b = s8 % NBUF
                pltpu.make_async_copy(x_hbm.at[pl.ds(0, C)],
                                      rows_bufs[b], gsems[b]).wait()
                pltpu.make_async_copy(ei_hbm.at[1, 0, 0],
                                      dst_bufs[s8], dsems[s8]).wait()
                pltpu.sync_copy(rows_bufs[b], acc_sh.at[dst_bufs[s8]],
                                add=True)
                pltpu.sync_copy(ones_v, deg_sh.at[dst_bufs[s8]], add=True)

                @pl.when(t + NBUF < NCHUNK)
                def _():
                    start_gather(t + NBUF, (s8 + NBUF) % NIDX, b)

                @pl.when(t + NIDX < NCHUNK)
                def _():
                    load_idx(t + NIDX, s8)

    plsc.subcore_barrier()

    pltpu.sync_copy(acc_sh.at[pl.ds(sid * RPS, RPS)],
                    part_hbm.at[cid, pl.ds(sid * RPS, RPS)])

    @pl.when(sid == 0)
    def _():
        pltpu.sync_copy(acc_sh.at[pl.ds(RPS * NS, TAIL)],
                        part_hbm.at[cid, pl.ds(RPS * NS, TAIL)])
        pltpu.sync_copy(deg_sh, deg_hbm.at[cid])


@functools.cache
def _make_aggregate():
    return pl.kernel(
        _agg_body,
        out_type=[
            jax.ShapeDtypeStruct((NC, N, D), jnp.float32),
            jax.ShapeDtypeStruct((NC, N), jnp.float32),
        ],
        mesh=plsc.VectorSubcoreMesh(core_axis_name="c", subcore_axis_name="s",
                                    num_cores=NC, num_subcores=NS),
        scratch_types=[
            [pltpu.VMEM((C, D), jnp.float32)] * NBUF,
            [pltpu.VMEM((C,), jnp.int32)] * NIDX,
            [pltpu.VMEM((C,), jnp.int32)] * NIDX,
            pltpu.VMEM((C,), jnp.float32),
            pltpu.VMEM_SHARED((N, D), jnp.float32),
            pltpu.VMEM_SHARED((N,), jnp.float32),
            [pltpu.SemaphoreType.DMA] * NBUF,
            [pltpu.SemaphoreType.DMA] * NIDX,
            [pltpu.SemaphoreType.DMA] * NIDX,
        ],
    )


BR = 1000


def _tc_body(p, d0, d1, x, wl, wr, b, o):
    deg = jnp.maximum(d0[...] + d1[...], 1.0)
    aggr = (p[0] + p[1]) / deg
    dims = (((1,), (1,)), ((), ()))
    o[...] = (lax.dot_general(aggr, wl[...], dims,
                              preferred_element_type=jnp.float32)
              + lax.dot_general(x[...], wr[...], dims,
                                preferred_element_type=jnp.float32)
              + b[...])


_combine = pl.pallas_call(
    _tc_body,
    grid=(N // BR,),
    in_specs=[
        pl.BlockSpec((2, BR, D), lambda i: (0, i, 0)),
        pl.BlockSpec((BR, 1), lambda i: (i, 0)),
        pl.BlockSpec((BR, 1), lambda i: (i, 0)),
        pl.BlockSpec((BR, D), lambda i: (i, 0)),
        pl.BlockSpec((D, D), lambda i: (0, 0)),
        pl.BlockSpec((D, D), lambda i: (0, 0)),
        pl.BlockSpec((1, D), lambda i: (0, 0)),
    ],
    out_specs=pl.BlockSpec((BR, D), lambda i: (i, 0)),
    out_shape=jax.ShapeDtypeStruct((N, D), jnp.float32),
)


def kernel(x, edge_index, W_l, W_r, b):
    ei = edge_index.astype(jnp.int32).reshape(2, NW, NCHUNK, C)
    zf = jnp.zeros((ZR, D), dtype=jnp.float32)
    zd = jnp.zeros((N,), dtype=jnp.float32)
    part, deg = _make_aggregate()(x, ei, zf, zd)
    return _combine(part, deg[0].reshape(N, 1), deg[1].reshape(N, 1),
                    x, W_l, W_r, b.reshape(1, D))
